# baseline (speedup 1.0000x reference)
import functools

import jax
import jax.numpy as jnp
from jax import lax
from jax.experimental import pallas as pl
from jax.experimental.pallas import tpu as pltpu
from jax.experimental.pallas import tpu_sc as plsc

_N = 100000
_D = 128


def _sc_body(lam_hbm, table_hbm, out_hbm, lam_v, row_v):
    cid = lax.axis_index("c")
    sid = lax.axis_index("s")

    @pl.when(jnp.logical_and(cid == 0, sid == 0))
    def _():
        pltpu.sync_copy(lam_hbm, lam_v)
        lam = lam_v[...][0]
        x = lam * jnp.float32(_N)
        idx = x.astype(jnp.int32)
        idx = jnp.where(idx.astype(jnp.float32) > x, idx - 1, idx)
        idx = jnp.clip(idx, 0, _N - 1)
        pltpu.sync_copy(table_hbm.at[pl.ds(idx, 1)], row_v)
        pltpu.sync_copy(row_v, out_hbm)


def kernel(lambd, intervals):
    lam16 = jnp.broadcast_to(jnp.asarray(lambd, jnp.float32), (16,))
    mesh = plsc.VectorSubcoreMesh(core_axis_name="c", subcore_axis_name="s")
    run = pl.kernel(
        _sc_body,
        mesh=mesh,
        out_type=jax.ShapeDtypeStruct((1, _D), jnp.float32),
        scratch_types=[
            pltpu.VMEM((16,), jnp.float32),
            pltpu.VMEM((1, _D), jnp.float32),
        ],
    )
    return run(lam16, intervals).reshape((_D,))


# speedup vs baseline: 1.0575x; 1.0575x over previous
import functools

import jax
import jax.numpy as jnp
from jax import lax
from jax.experimental import pallas as pl
from jax.experimental.pallas import tpu as pltpu
from jax.experimental.pallas import tpu_sc as plsc

_N = 100000
_D = 128


def _sc_body(lam_hbm, table_hbm, out_hbm, lam_v, row_v):
    cid = lax.axis_index("c")
    sid = lax.axis_index("s")

    @pl.when(jnp.logical_and(cid == 0, sid == 0))
    def _():
        pltpu.sync_copy(lam_hbm, lam_v)
        lam = lam_v[...][0]
        x = lam * jnp.float32(_N)
        idx = x.astype(jnp.int32)
        idx = jnp.where(idx.astype(jnp.float32) > x, idx - 1, idx)
        idx = jnp.clip(idx, 0, _N - 1)
        pltpu.sync_copy(table_hbm.at[pl.ds(idx, 1)], row_v)
        pltpu.sync_copy(row_v, out_hbm)


def kernel(lambd, intervals):
    lam16 = jnp.broadcast_to(jnp.asarray(lambd, jnp.float32), (16,))
    mesh = plsc.VectorSubcoreMesh(core_axis_name="c", subcore_axis_name="s", num_cores=1)
    run = pl.kernel(
        _sc_body,
        mesh=mesh,
        out_type=jax.ShapeDtypeStruct((1, _D), jnp.float32),
        scratch_types=[
            pltpu.VMEM((16,), jnp.float32),
            pltpu.VMEM((1, _D), jnp.float32),
        ],
    )
    return run(lam16, intervals).reshape((_D,))


# speedup vs baseline: 1.1770x; 1.1130x over previous
import jax
import jax.numpy as jnp
from jax import lax
from jax.experimental import pallas as pl
from jax.experimental.pallas import tpu as pltpu
from jax.experimental.pallas import tpu_sc as plsc

_N = 100000
_D = 128


def _scs_body(lam_hbm, table_hbm, out_hbm, lam_s):
    cid = lax.axis_index("c")

    @pl.when(cid == 0)
    def _():
        pltpu.sync_copy(lam_hbm, lam_s)
        lam = lam_s[0]
        x = lam * jnp.float32(_N)
        idx = x.astype(jnp.int32)
        idx = jnp.where(idx.astype(jnp.float32) > x, idx - 1, idx)
        idx = jnp.clip(idx, 0, _N - 1)
        pltpu.sync_copy(table_hbm.at[pl.ds(idx, 1)], out_hbm)


def kernel(lambd, intervals):
    lam16 = jnp.broadcast_to(jnp.asarray(lambd, jnp.float32), (16,))
    mesh = plsc.ScalarSubcoreMesh(axis_name="c", num_cores=1)
    run = pl.kernel(
        _scs_body,
        mesh=mesh,
        out_type=jax.ShapeDtypeStruct((1, _D), jnp.float32),
        scratch_types=[
            pltpu.SMEM((16,), jnp.float32),
        ],
    )
    return run(lam16, intervals).reshape((_D,))
